# baseline (device time: 227607 ns/iter reference)
import jax
import jax.numpy as jnp
from jax import lax
from jax.experimental import pallas as pl
from jax.experimental.pallas import tpu as pltpu

N_DEV = 16
NSLOTS = 4
SUB = 2

RING = [0, 1, 5, 4, 8, 9, 13, 12, 15, 14, 10, 11, 7, 6, 2, 3]
INV = [0] * N_DEV
for _r, _lg in enumerate(RING):
    INV[_lg] = _r


def kernel(x, w_mat, scale_x, scale_w):
    m, _ = x.shape
    _, n = w_mat.shape
    ch = m // N_DEV
    hn = n // 2
    sb = ch // SUB

    my_log = lax.axis_index("i")
    ring_arr = jnp.asarray(RING, dtype=jnp.int32)
    inv_arr = jnp.asarray(INV, dtype=jnp.int32)
    kpos = inv_arr[my_log]
    right_log = ring_arr[(kpos + 1) % N_DEV]
    left_log = ring_arr[(kpos + N_DEV - 1) % N_DEV]
    pos = jnp.stack([kpos, left_log, right_log]).astype(jnp.int32)
    pos = pos.reshape(3, 1)

    def body(x_ref, w_ref, sx_ref, sw_ref, pos_ref, out_ref, part_ref,
             rs_comm_cw, rs_comm_ccw, ag_comm_cw, ag_comm_ccw,
             rs_send_cw, rs_recv_cw, rs_send_ccw, rs_recv_ccw,
             ag_send_cw, ag_recv_cw, ag_send_ccw, ag_recv_ccw,
             credits):
        my = pos_ref[0, 0]
        left = pos_ref[1, 0]
        right = pos_ref[2, 0]

        part_ref[pl.ds(my * ch, ch), :] = jnp.dot(
            x_ref[pl.ds(my * ch, ch), :], w_ref[...],
            preferred_element_type=jnp.float32,
        ).astype(jnp.bfloat16)

        barrier = pltpu.get_barrier_semaphore()
        for nbr in (left, right):
            pl.semaphore_signal(barrier, inc=1, device_id=(nbr,),
                                device_id_type=pl.DeviceIdType.MESH)
        pl.semaphore_wait(barrier, 2)

        def subrows(c, b):
            return pl.ds(c * ch + b * sb, sb)

        flows = []
        for b in range(SUB):
            flows.append(dict(
                b=b, half=pl.ds(0, hn), to=right, sgn=+1,
                comm=rs_comm_cw, send=rs_send_cw, recv=rs_recv_cw,
                agcomm=ag_comm_cw, agsend=ag_send_cw, agrecv=ag_recv_cw,
                rs_credit=0 * SUB + b, ag_credit=2 * SUB + b,
                credit_to=left,
            ))
            flows.append(dict(
                b=b, half=pl.ds(hn, hn), to=left, sgn=-1,
                comm=rs_comm_ccw, send=rs_send_ccw, recv=rs_recv_ccw,
                agcomm=ag_comm_ccw, agsend=ag_send_ccw, agrecv=ag_recv_ccw,
                rs_credit=1 * SUB + b, ag_credit=3 * SUB + b,
                credit_to=right,
            ))

        def rs_rdma(f, s):
            c = (my + f["sgn"] * (-s) + N_DEV) % N_DEV
            return pltpu.make_async_remote_copy(
                src_ref=part_ref.at[subrows(c, f["b"]), f["half"]],
                dst_ref=f["comm"].at[f["b"], s % NSLOTS],
                send_sem=f["send"].at[f["b"], s],
                recv_sem=f["recv"].at[f["b"], s],
                device_id=(f["to"],),
                device_id_type=pl.DeviceIdType.MESH,
            )

        for f in flows:
            rs_rdma(f, 0).start()
        for s in range(N_DEV - 1):
            for f in flows:
                rs_rdma(f, s).wait()
                ac = (my + f["sgn"] * (-s - 1) + N_DEV) % N_DEV
                part_ref[subrows(ac, f["b"]), f["half"]] = (
                    jnp.dot(x_ref[subrows(ac, f["b"]), :],
                            w_ref[:, f["half"]],
                            preferred_element_type=jnp.float32)
                    + f["comm"][f["b"], s % NSLOTS].astype(jnp.float32)
                ).astype(jnp.bfloat16)
                if s < N_DEV - 2:
                    if s + 1 >= NSLOTS:
                        pl.semaphore_wait(credits.at[f["rs_credit"]], 1)
                    rs_rdma(f, s + 1).start()
                if s < (N_DEV - 1) - NSLOTS:
                    pl.semaphore_signal(
                        credits.at[f["rs_credit"]], inc=1,
                        device_id=(f["credit_to"],),
                        device_id_type=pl.DeviceIdType.MESH)

        scale = sx_ref[0, 0] * sw_ref[0, 0]
        own_cw = (my + 1) % N_DEV
        own_ccw = (my - 1 + N_DEV) % N_DEV
        for own, half in ((own_cw, pl.ds(0, hn)), (own_ccw, pl.ds(hn, hn))):
            rws = pl.ds(own * ch, ch)
            v = jnp.maximum(
                part_ref[rws, half].astype(jnp.float32) * scale, 0.0)
            out_ref[rws, half] = v
            part_ref[rws, half] = v.astype(jnp.bfloat16)

        def ag_rdma(f, s):
            own = (my + f["sgn"] + N_DEV) % N_DEV
            if s == 0:
                src = part_ref.at[subrows(own, f["b"]), f["half"]]
            else:
                src = f["agcomm"].at[f["b"], (s - 1) % NSLOTS]
            return pltpu.make_async_remote_copy(
                src_ref=src,
                dst_ref=f["agcomm"].at[f["b"], s % NSLOTS],
                send_sem=f["agsend"].at[f["b"], s],
                recv_sem=f["agrecv"].at[f["b"], s],
                device_id=(f["to"],),
                device_id_type=pl.DeviceIdType.MESH,
            )

        for f in flows:
            ag_rdma(f, 0).start()
        for s in range(N_DEV - 1):
            for f in flows:
                ag_rdma(f, s).wait()
                if s < N_DEV - 2:
                    if s + 1 >= NSLOTS:
                        pl.semaphore_wait(credits.at[f["ag_credit"]], 1)
                    ag_rdma(f, s + 1).start()
                gc = (my + f["sgn"] * (-s) + N_DEV) % N_DEV
                out_ref[subrows(gc, f["b"]), f["half"]] = (
                    f["agcomm"][f["b"], s % NSLOTS].astype(jnp.float32))
                if 1 <= s <= (N_DEV - 1) - NSLOTS:
                    pl.semaphore_signal(
                        credits.at[f["ag_credit"]], inc=1,
                        device_id=(f["credit_to"],),
                        device_id_type=pl.DeviceIdType.MESH)

    dma2 = pltpu.SemaphoreType.DMA((SUB, N_DEV - 1))
    return pl.pallas_call(
        body,
        out_shape=jax.ShapeDtypeStruct((m, n), jnp.float32),
        in_specs=[
            pl.BlockSpec(memory_space=pltpu.VMEM),
            pl.BlockSpec(memory_space=pltpu.VMEM),
            pl.BlockSpec(memory_space=pltpu.SMEM),
            pl.BlockSpec(memory_space=pltpu.SMEM),
            pl.BlockSpec(memory_space=pltpu.SMEM),
        ],
        out_specs=pl.BlockSpec(memory_space=pltpu.VMEM),
        scratch_shapes=[
            pltpu.VMEM((m, n), jnp.bfloat16),
            pltpu.VMEM((SUB, NSLOTS, sb, hn), jnp.bfloat16),
            pltpu.VMEM((SUB, NSLOTS, sb, hn), jnp.bfloat16),
            pltpu.VMEM((SUB, NSLOTS, sb, hn), jnp.bfloat16),
            pltpu.VMEM((SUB, NSLOTS, sb, hn), jnp.bfloat16),
            dma2, dma2, dma2, dma2,
            dma2, dma2, dma2, dma2,
            pltpu.SemaphoreType.REGULAR((4 * SUB,)),
        ],
        compiler_params=pltpu.CompilerParams(
            collective_id=0,
            vmem_limit_bytes=120 * 1024 * 1024,
        ),
    )(x.astype(jnp.bfloat16), w_mat.astype(jnp.bfloat16),
      scale_x.reshape(1, 1), scale_w.reshape(1, 1), pos)


# device time: 227104 ns/iter; 1.0022x vs baseline; 1.0022x over previous
import os

import jax
import jax.numpy as jnp
from jax import lax
from jax.experimental import pallas as pl
from jax.experimental.pallas import tpu as pltpu

PHASES = os.environ.get("KERNEL_PHASES", "rs,ag")

N_DEV = 16
NSLOTS = 4
SUB = 2

RING = [0, 1, 5, 4, 8, 9, 13, 12, 15, 14, 10, 11, 7, 6, 2, 3]
INV = [0] * N_DEV
for _r, _lg in enumerate(RING):
    INV[_lg] = _r


def kernel(x, w_mat, scale_x, scale_w):
    m, _ = x.shape
    _, n = w_mat.shape
    ch = m // N_DEV
    hn = n // 2
    sb = ch // SUB

    my_log = lax.axis_index("i")
    ring_arr = jnp.asarray(RING, dtype=jnp.int32)
    inv_arr = jnp.asarray(INV, dtype=jnp.int32)
    kpos = inv_arr[my_log]
    right_log = ring_arr[(kpos + 1) % N_DEV]
    left_log = ring_arr[(kpos + N_DEV - 1) % N_DEV]
    pos = jnp.stack([kpos, left_log, right_log]).astype(jnp.int32)
    pos = pos.reshape(3, 1)

    def body(x_ref, w_ref, sx_ref, sw_ref, pos_ref, out_ref, part_ref,
             rs_comm_cw, rs_comm_ccw, ag_comm_cw, ag_comm_ccw,
             rs_send_cw, rs_recv_cw, rs_send_ccw, rs_recv_ccw,
             ag_send_cw, ag_recv_cw, ag_send_ccw, ag_recv_ccw,
             credits):
        my = pos_ref[0, 0]
        left = pos_ref[1, 0]
        right = pos_ref[2, 0]

        part_ref[pl.ds(my * ch, ch), :] = jnp.dot(
            x_ref[pl.ds(my * ch, ch), :], w_ref[...],
            preferred_element_type=jnp.float32,
        ).astype(jnp.bfloat16)

        barrier = pltpu.get_barrier_semaphore()
        for nbr in (left, right):
            pl.semaphore_signal(barrier, inc=1, device_id=(nbr,),
                                device_id_type=pl.DeviceIdType.MESH)
        pl.semaphore_wait(barrier, 2)

        def subrows(c, b):
            return pl.ds(c * ch + b * sb, sb)

        flows = []
        for b in range(SUB):
            flows.append(dict(
                b=b, half=pl.ds(0, hn), h0=0, to=right, sgn=+1,
                comm=rs_comm_cw, send=rs_send_cw, recv=rs_recv_cw,
                agcomm=ag_comm_cw, agsend=ag_send_cw, agrecv=ag_recv_cw,
                rs_credit=0 * SUB + b, ag_credit=2 * SUB + b,
                credit_to=left,
            ))
            flows.append(dict(
                b=b, half=pl.ds(hn, hn), h0=hn, to=left, sgn=-1,
                comm=rs_comm_ccw, send=rs_send_ccw, recv=rs_recv_ccw,
                agcomm=ag_comm_ccw, agsend=ag_send_ccw, agrecv=ag_recv_ccw,
                rs_credit=1 * SUB + b, ag_credit=3 * SUB + b,
                credit_to=right,
            ))

        wn = hn // 2 if "thin" in PHASES else hn

        def rs_rdma(f, s):
            c = (my + f["sgn"] * (-s) + N_DEV) % N_DEV
            return pltpu.make_async_remote_copy(
                src_ref=part_ref.at[subrows(c, f["b"]), pl.ds(f["h0"], wn)],
                dst_ref=f["comm"].at[f["b"], s % NSLOTS],
                send_sem=f["send"].at[f["b"], s],
                recv_sem=f["recv"].at[f["b"], s],
                device_id=(f["to"],),
                device_id_type=pl.DeviceIdType.MESH,
            )

        for f in flows:
            if "rs" not in PHASES:
                break
            rs_rdma(f, 0).start()
        for s in range(N_DEV - 1 if "rs" in PHASES else 0):
            for f in flows:
                rs_rdma(f, s).wait()
                if "noacc" not in PHASES:
                    ac = (my + f["sgn"] * (-s - 1) + N_DEV) % N_DEV
                    part_ref[subrows(ac, f["b"]), f["half"]] = (
                        jnp.dot(x_ref[subrows(ac, f["b"]), :],
                                w_ref[:, f["half"]],
                                preferred_element_type=jnp.float32)
                        + f["comm"][f["b"], s % NSLOTS].astype(jnp.float32)
                    ).astype(jnp.bfloat16)
                if s < N_DEV - 2:
                    rs_rdma(f, s + 1).start()

        scale = sx_ref[0, 0] * sw_ref[0, 0]
        own_cw = (my + 1) % N_DEV
        own_ccw = (my - 1 + N_DEV) % N_DEV
        for own, half in ((own_cw, pl.ds(0, hn)), (own_ccw, pl.ds(hn, hn))):
            rws = pl.ds(own * ch, ch)
            v = jnp.maximum(
                part_ref[rws, half].astype(jnp.float32) * scale, 0.0)
            out_ref[rws, half] = v
            part_ref[rws, half] = v.astype(jnp.bfloat16)

        def ag_rdma(f, s):
            own = (my + f["sgn"] + N_DEV) % N_DEV
            if s == 0:
                src = part_ref.at[subrows(own, f["b"]), f["half"]]
            else:
                src = f["agcomm"].at[f["b"], (s - 1) % NSLOTS]
            return pltpu.make_async_remote_copy(
                src_ref=src,
                dst_ref=f["agcomm"].at[f["b"], s % NSLOTS],
                send_sem=f["agsend"].at[f["b"], s],
                recv_sem=f["agrecv"].at[f["b"], s],
                device_id=(f["to"],),
                device_id_type=pl.DeviceIdType.MESH,
            )

        for f in flows:
            if "ag" not in PHASES:
                break
            ag_rdma(f, 0).start()
        for s in range(N_DEV - 1 if "ag" in PHASES else 0):
            for f in flows:
                ag_rdma(f, s).wait()
                if s < N_DEV - 2:
                    ag_rdma(f, s + 1).start()
                if "noconv" not in PHASES:
                    gc = (my + f["sgn"] * (-s) + N_DEV) % N_DEV
                    out_ref[subrows(gc, f["b"]), f["half"]] = (
                        f["agcomm"][f["b"], s % NSLOTS].astype(jnp.float32))


    dma2 = pltpu.SemaphoreType.DMA((SUB, N_DEV - 1))
    return pl.pallas_call(
        body,
        out_shape=jax.ShapeDtypeStruct((m, n), jnp.float32),
        in_specs=[
            pl.BlockSpec(memory_space=pltpu.VMEM),
            pl.BlockSpec(memory_space=pltpu.VMEM),
            pl.BlockSpec(memory_space=pltpu.SMEM),
            pl.BlockSpec(memory_space=pltpu.SMEM),
            pl.BlockSpec(memory_space=pltpu.SMEM),
        ],
        out_specs=pl.BlockSpec(memory_space=pltpu.VMEM),
        scratch_shapes=[
            pltpu.VMEM((m, n), jnp.bfloat16),
            pltpu.VMEM((SUB, NSLOTS, sb, hn // 2 if "thin" in PHASES else hn),
                       jnp.bfloat16),
            pltpu.VMEM((SUB, NSLOTS, sb, hn // 2 if "thin" in PHASES else hn),
                       jnp.bfloat16),
            pltpu.VMEM((SUB, NSLOTS, sb, hn), jnp.bfloat16),
            pltpu.VMEM((SUB, NSLOTS, sb, hn), jnp.bfloat16),
            dma2, dma2, dma2, dma2,
            dma2, dma2, dma2, dma2,
            pltpu.SemaphoreType.REGULAR((4 * SUB,)),
        ],
        compiler_params=pltpu.CompilerParams(
            collective_id=0,
            vmem_limit_bytes=120 * 1024 * 1024,
        ),
    )(x.astype(jnp.bfloat16), w_mat.astype(jnp.bfloat16),
      scale_x.reshape(1, 1), scale_w.reshape(1, 1), pos)


# device time: 226574 ns/iter; 1.0046x vs baseline; 1.0023x over previous
import jax
import jax.numpy as jnp
from jax import lax
from jax.experimental import pallas as pl
from jax.experimental.pallas import tpu as pltpu

N_DEV = 16
NSLOTS = 4
SUB = 2

RING = [0, 1, 5, 4, 8, 9, 13, 12, 15, 14, 10, 11, 7, 6, 2, 3]
INV = [0] * N_DEV
for _r, _lg in enumerate(RING):
    INV[_lg] = _r


def kernel(x, w_mat, scale_x, scale_w):
    m, _ = x.shape
    _, n = w_mat.shape
    ch = m // N_DEV
    hn = n // 2
    sb = ch // SUB

    my_log = lax.axis_index("i")
    ring_arr = jnp.asarray(RING, dtype=jnp.int32)
    inv_arr = jnp.asarray(INV, dtype=jnp.int32)
    kpos = inv_arr[my_log]
    right_log = ring_arr[(kpos + 1) % N_DEV]
    left_log = ring_arr[(kpos + N_DEV - 1) % N_DEV]
    pos = jnp.stack([kpos, left_log, right_log]).astype(jnp.int32)
    pos = pos.reshape(3, 1)

    def body(x_ref, w_ref, sx_ref, sw_ref, pos_ref, out_ref, part_ref,
             rs_comm_cw, rs_comm_ccw, ag_comm_cw, ag_comm_ccw,
             rs_send_cw, rs_recv_cw, rs_send_ccw, rs_recv_ccw,
             ag_send_cw, ag_recv_cw, ag_send_ccw, ag_recv_ccw):
        my = pos_ref[0, 0]
        left = pos_ref[1, 0]
        right = pos_ref[2, 0]

        part_ref[pl.ds(my * ch, ch), :] = jnp.dot(
            x_ref[pl.ds(my * ch, ch), :], w_ref[...],
            preferred_element_type=jnp.float32,
        ).astype(jnp.bfloat16)

        barrier = pltpu.get_barrier_semaphore()
        for nbr in (left, right):
            pl.semaphore_signal(barrier, inc=1, device_id=(nbr,),
                                device_id_type=pl.DeviceIdType.MESH)
        pl.semaphore_wait(barrier, 2)

        def subrows(c, b):
            return pl.ds(c * ch + b * sb, sb)

        cw_flows, ccw_flows = [], []
        for b in range(SUB):
            cw_flows.append(dict(
                b=b, half=pl.ds(0, hn), to=right, sgn=+1,
                comm=rs_comm_cw, send=rs_send_cw, recv=rs_recv_cw,
                agcomm=ag_comm_cw, agsend=ag_send_cw, agrecv=ag_recv_cw,
            ))
            ccw_flows.append(dict(
                b=b, half=pl.ds(hn, hn), to=left, sgn=-1,
                comm=rs_comm_ccw, send=rs_send_ccw, recv=rs_recv_ccw,
                agcomm=ag_comm_ccw, agsend=ag_send_ccw, agrecv=ag_recv_ccw,
            ))
        flows = [fl for pair in zip(cw_flows, ccw_flows) for fl in pair]

        def rs_rdma(f, s):
            c = (my + f["sgn"] * (-s) + N_DEV) % N_DEV
            return pltpu.make_async_remote_copy(
                src_ref=part_ref.at[subrows(c, f["b"]), f["half"]],
                dst_ref=f["comm"].at[f["b"], s % NSLOTS],
                send_sem=f["send"].at[f["b"], s],
                recv_sem=f["recv"].at[f["b"], s],
                device_id=(f["to"],),
                device_id_type=pl.DeviceIdType.MESH,
            )

        def rs_step(f, s):
            rs_rdma(f, s).wait()
            ac = (my + f["sgn"] * (-s - 1) + N_DEV) % N_DEV
            part_ref[subrows(ac, f["b"]), f["half"]] = (
                jnp.dot(x_ref[subrows(ac, f["b"]), :], w_ref[:, f["half"]],
                        preferred_element_type=jnp.float32)
                + f["comm"][f["b"], s % NSLOTS].astype(jnp.float32)
            ).astype(jnp.bfloat16)
            if s < N_DEV - 2:
                rs_rdma(f, s + 1).start()

        def ag_rdma(f, s):
            own = (my + f["sgn"] + N_DEV) % N_DEV
            if s == 0:
                src = part_ref.at[subrows(own, f["b"]), f["half"]]
            else:
                src = f["agcomm"].at[f["b"], (s - 1) % NSLOTS]
            return pltpu.make_async_remote_copy(
                src_ref=src,
                dst_ref=f["agcomm"].at[f["b"], s % NSLOTS],
                send_sem=f["agsend"].at[f["b"], s],
                recv_sem=f["agrecv"].at[f["b"], s],
                device_id=(f["to"],),
                device_id_type=pl.DeviceIdType.MESH,
            )

        scale = sx_ref[0, 0] * sw_ref[0, 0]

        def epilogue_and_ag0(dir_flows):
            f0 = dir_flows[0]
            own = (my + f0["sgn"] + N_DEV) % N_DEV
            rws = pl.ds(own * ch, ch)
            v = jnp.maximum(
                part_ref[rws, f0["half"]].astype(jnp.float32) * scale, 0.0)
            out_ref[rws, f0["half"]] = v
            part_ref[rws, f0["half"]] = v.astype(jnp.bfloat16)
            for f in dir_flows:
                ag_rdma(f, 0).start()

        for f in flows:
            rs_rdma(f, 0).start()
        for s in range(N_DEV - 2):
            for f in flows:
                rs_step(f, s)
        for dir_flows in (cw_flows, ccw_flows):
            for f in dir_flows:
                rs_step(f, N_DEV - 2)
            epilogue_and_ag0(dir_flows)

        for s in range(N_DEV - 1):
            for f in flows:
                ag_rdma(f, s).wait()
                if s < N_DEV - 2:
                    ag_rdma(f, s + 1).start()
                gc = (my + f["sgn"] * (-s) + N_DEV) % N_DEV
                out_ref[subrows(gc, f["b"]), f["half"]] = (
                    f["agcomm"][f["b"], s % NSLOTS].astype(jnp.float32))

    dma2 = pltpu.SemaphoreType.DMA((SUB, N_DEV - 1))
    return pl.pallas_call(
        body,
        out_shape=jax.ShapeDtypeStruct((m, n), jnp.float32),
        in_specs=[
            pl.BlockSpec(memory_space=pltpu.VMEM),
            pl.BlockSpec(memory_space=pltpu.VMEM),
            pl.BlockSpec(memory_space=pltpu.SMEM),
            pl.BlockSpec(memory_space=pltpu.SMEM),
            pl.BlockSpec(memory_space=pltpu.SMEM),
        ],
        out_specs=pl.BlockSpec(memory_space=pltpu.VMEM),
        scratch_shapes=[
            pltpu.VMEM((m, n), jnp.bfloat16),
            pltpu.VMEM((SUB, NSLOTS, sb, hn), jnp.bfloat16),
            pltpu.VMEM((SUB, NSLOTS, sb, hn), jnp.bfloat16),
            pltpu.VMEM((SUB, NSLOTS, sb, hn), jnp.bfloat16),
            pltpu.VMEM((SUB, NSLOTS, sb, hn), jnp.bfloat16),
            dma2, dma2, dma2, dma2,
            dma2, dma2, dma2, dma2,
        ],
        compiler_params=pltpu.CompilerParams(
            collective_id=0,
            vmem_limit_bytes=120 * 1024 * 1024,
        ),
    )(x.astype(jnp.bfloat16), w_mat.astype(jnp.bfloat16),
      scale_x.reshape(1, 1), scale_w.reshape(1, 1), pos)
